# baseline (speedup 1.0000x reference)
import functools

import jax
import jax.numpy as jnp
from jax import lax
from jax.experimental import pallas as pl
from jax.experimental.pallas import tpu as pltpu
from jax.experimental.pallas import tpu_sc as plsc

K_TOP = 200
NMS_THR = 0.5
B = 4
N = 5000
NP = 5120
KP = 208
L = 16
NC = 2
NW = 32


def _nms_body(box_h, ord_h,
              keep_h, ret_h,
              vbox, vord,
              ky1, kx1, ky2, kx2, kar,
              okeep, oret,
              kcnt):
    c = lax.axis_index("c")
    s = lax.axis_index("s")
    wid = s * NC + c
    b = jnp.minimum(wid, B - 1)

    pltpu.sync_copy(box_h.at[b], vbox)
    pltpu.sync_copy(ord_h.at[b], vord)

    zf = jnp.zeros((L,), jnp.float32)
    zi = jnp.zeros((L,), jnp.int32)
    for t in range(KP // L):
        sl = pl.ds(t * L, L)
        ky1[sl] = zf
        kx1[sl] = zf
        ky2[sl] = zf
        kx2[sl] = zf
        kar[sl] = zf
        okeep[sl] = zi
        for cc in range(4):
            oret[cc, sl] = zf

    lanes = lax.iota(jnp.int32, L)
    lane0 = lanes == 0
    c0 = jnp.zeros((L,), jnp.int32)
    c1 = jnp.full((L,), 1, jnp.int32)
    c2 = jnp.full((L,), 2, jnp.int32)
    c3 = jnp.full((L,), 3, jnp.int32)

    kcnt[0] = jnp.int32(0)

    def pos_body(ordb, y1b, x1b, y2b, x2b, j, carry):
        kept = kcnt[0]
        jv = jnp.full((L,), j, jnp.int32)
        idxv = ordb.at[jv].get(mode="promise_in_bounds")
        y1c = y1b.at[jv].get(mode="promise_in_bounds")
        x1c = x1b.at[jv].get(mode="promise_in_bounds")
        y2c = y2b.at[jv].get(mode="promise_in_bounds")
        x2c = x2b.at[jv].get(mode="promise_in_bounds")
        areac = (x2c - x1c) * (y2c - y1c)
        elig = jnp.logical_and(jnp.max(areac) >= 4.0, kept < K_TOP)

        nk = (kept + (L - 1)) // L

        def iou_step(t, miou):
            sl = pl.ds(t * L, L)
            a1 = ky1[sl]
            b1 = kx1[sl]
            a2 = ky2[sl]
            b2 = kx2[sl]
            ka = kar[sl]
            q_y1 = jnp.minimum(jnp.maximum(y1c, a1), a2)
            q_x1 = jnp.minimum(jnp.maximum(x1c, b1), b2)
            q_y2 = jnp.minimum(jnp.maximum(y2c, a1), a2)
            q_x2 = jnp.minimum(jnp.maximum(x2c, b1), b2)
            inter = (q_x2 - q_x1) * (q_y2 - q_y1)
            union = areac + ka - inter
            return jnp.maximum(miou, inter / union)

        miou = lax.fori_loop(0, nk, iou_step,
                             jnp.full((L,), -1.0, jnp.float32))
        take = jnp.logical_and(elig, jnp.max(miou) <= NMS_THR)

        @pl.when(take)
        def _accept():
            kv = jnp.full((L,), kept, jnp.int32)
            plsc.store_scatter(ky1, [kv], y1c, mask=lane0)
            plsc.store_scatter(kx1, [kv], x1c, mask=lane0)
            plsc.store_scatter(ky2, [kv], y2c, mask=lane0)
            plsc.store_scatter(kx2, [kv], x2c, mask=lane0)
            plsc.store_scatter(kar, [kv], areac, mask=lane0)
            plsc.store_scatter(okeep, [kv], idxv, mask=lane0)
            plsc.store_scatter(oret, [c0, kv], y1c, mask=lane0)
            plsc.store_scatter(oret, [c1, kv], x1c, mask=lane0)
            plsc.store_scatter(oret, [c2, kv], y2c, mask=lane0)
            plsc.store_scatter(oret, [c3, kv], x2c, mask=lane0)
            kcnt[0] = kept + 1

        return carry

    def blk_body(t, carry):
        @pl.when(kcnt[0] < K_TOP)
        def _blk():
            sl = pl.ds(t * L, L)
            ordb = vord[sl]
            fb = ordb * 4
            y1b = plsc.load_gather(vbox, [fb])
            x1b = plsc.load_gather(vbox, [fb + c1])
            y2b = plsc.load_gather(vbox, [fb + c2])
            x2b = plsc.load_gather(vbox, [fb + c3])
            lax.fori_loop(0, L,
                          functools.partial(pos_body, ordb, y1b, x1b,
                                            y2b, x2b),
                          jnp.int32(0))
        return carry

    lax.fori_loop(0, NP // L, blk_body, jnp.int32(0))

    plsc.store_scatter(okeep, [jnp.full((L,), K_TOP, jnp.int32)],
                       jnp.full((L,), kcnt[0], jnp.int32), mask=lane0)

    pltpu.sync_copy(okeep, keep_h.at[wid])
    pltpu.sync_copy(oret, ret_h.at[wid])


_nms_sc = functools.partial(
    pl.kernel,
    out_type=(
        jax.ShapeDtypeStruct((NW, KP), jnp.int32),
        jax.ShapeDtypeStruct((NW, 4, KP), jnp.float32),
    ),
    mesh=plsc.VectorSubcoreMesh(core_axis_name="c", subcore_axis_name="s"),
    scratch_types=[
        pltpu.VMEM((NP * 4,), jnp.float32),
        pltpu.VMEM((NP,), jnp.int32),
        pltpu.VMEM((KP,), jnp.float32),
        pltpu.VMEM((KP,), jnp.float32),
        pltpu.VMEM((KP,), jnp.float32),
        pltpu.VMEM((KP,), jnp.float32),
        pltpu.VMEM((KP,), jnp.float32),
        pltpu.VMEM((KP,), jnp.int32),
        pltpu.VMEM((4, KP), jnp.float32),
        pltpu.SMEM((1,), jnp.int32),
    ],
    compiler_params=pltpu.CompilerParams(needs_layout_passes=False),
)(_nms_body)


def kernel(scoress, bboxess):
    probs = jax.nn.softmax(scoress, axis=2)
    sc = probs[:, :, 0]
    order_desc = jnp.argsort(sc, axis=1, stable=True)[:, ::-1].astype(jnp.int32)

    orderp = jnp.pad(order_desc, ((0, 0), (0, NP - N)), constant_values=N)
    boxp = jnp.pad(bboxess, ((0, 0), (0, NP - N), (0, 0))).reshape(B, NP * 4)

    okeep, oret = _nms_sc(boxp, orderp)

    keeps = okeep[:B, :K_TOP].astype(jnp.int64)
    counts = okeep[:B, K_TOP:K_TOP + 1].astype(jnp.int64)
    ret = jnp.transpose(oret[:B, :, :K_TOP], (0, 2, 1))
    return (ret, counts, keeps)
